# baseline (speedup 1.0000x reference)
import jax
import jax.numpy as jnp
from jax import lax
from jax.experimental import pallas as pl
from jax.experimental.pallas import tpu as pltpu

N = 20000
K = 1024
BN = 2000

_DNUMS = (((1,), (1,)), ((), ()))


def _heads_kernel(x_ref, wc_ref, wb_ref, s_ref, d_ref):
    xb = x_ref[...].astype(jnp.bfloat16)
    wc = wc_ref[...].astype(jnp.bfloat16)
    wb = wb_ref[...].astype(jnp.bfloat16)
    s_ref[...] = lax.dot_general(xb, wc, _DNUMS, preferred_element_type=jnp.float32)
    d_ref[...] = lax.dot_general(xb, wb, _DNUMS, preferred_element_type=jnp.float32)


def kernel(x, W_cls, b_cls, W_bbox, b_bbox):
    wcp = jnp.zeros((128, K), jnp.float32).at[:82].set(W_cls)
    wbp = jnp.zeros((384, K), jnp.float32).at[:324].set(W_bbox)
    grid = (N // BN,)
    scores, deltas = pl.pallas_call(
        _heads_kernel,
        grid=grid,
        in_specs=[
            pl.BlockSpec((BN, K), lambda i: (i, 0)),
            pl.BlockSpec((128, K), lambda i: (0, 0)),
            pl.BlockSpec((384, K), lambda i: (0, 0)),
        ],
        out_specs=[
            pl.BlockSpec((BN, 128), lambda i: (i, 0)),
            pl.BlockSpec((BN, 384), lambda i: (i, 0)),
        ],
        out_shape=[
            jax.ShapeDtypeStruct((N, 128), jnp.float32),
            jax.ShapeDtypeStruct((N, 384), jnp.float32),
        ],
        compiler_params=pltpu.CompilerParams(
            dimension_semantics=("arbitrary",),
        ),
    )(x, wcp, wbp)
    return scores[:, :82] + b_cls, deltas[:, :324] + b_bbox


# speedup vs baseline: 2.5681x; 2.5681x over previous
import jax
import jax.numpy as jnp
from jax import lax
from jax.experimental import pallas as pl
from jax.experimental.pallas import tpu as pltpu

N = 20000
K = 1024
C_CLS = 82
C_BOX = 324
BN = 2000
STEPS = N // BN
NBUF = 4

_DNUMS = (((1,), (1,)), ((), ()))


def _heads_kernel(x_ref, wc_ref, wb_ref, bc_ref, bb_ref, s_hbm, d_hbm,
                  s_scr, d_scr, s_sem, d_sem):
    i = pl.program_id(0)
    slot = lax.rem(i, NBUF)

    def s_copy(step, sl):
        return pltpu.make_async_copy(
            s_scr.at[sl], s_hbm.at[pl.ds(step * BN, BN), :], s_sem.at[sl])

    def d_copy(step, sl):
        return pltpu.make_async_copy(
            d_scr.at[sl], d_hbm.at[pl.ds(step * BN, BN), :], d_sem.at[sl])

    @pl.when(i >= NBUF)
    def _reclaim():
        s_copy(i - NBUF, slot).wait()
        d_copy(i - NBUF, slot).wait()

    xb = x_ref[...].astype(jnp.bfloat16)
    wc = wc_ref[...].astype(jnp.bfloat16)
    wb = wb_ref[...].astype(jnp.bfloat16)
    s = lax.dot_general(xb, wc, _DNUMS, preferred_element_type=jnp.float32)
    d = lax.dot_general(xb, wb, _DNUMS, preferred_element_type=jnp.float32)
    s_scr[slot] = s + bc_ref[...]
    d_scr[slot] = d + bb_ref[...]

    s_copy(i, slot).start()
    d_copy(i, slot).start()

    @pl.when(i == STEPS - 1)
    def _drain():
        for back in range(min(NBUF, STEPS) - 1, -1, -1):
            step = STEPS - 1 - back
            s_copy(step, lax.rem(step, NBUF)).wait()
            d_copy(step, lax.rem(step, NBUF)).wait()


def kernel(x, W_cls, b_cls, W_bbox, b_bbox):
    if x.ndim > 2:
        x = x.reshape(x.shape[0], -1)
    bc = b_cls.reshape(1, C_CLS)
    bb = b_bbox.reshape(1, C_BOX)

    scores, deltas = pl.pallas_call(
        _heads_kernel,
        grid=(STEPS,),
        in_specs=[
            pl.BlockSpec((BN, K), lambda i: (i, 0)),
            pl.BlockSpec((C_CLS, K), lambda i: (0, 0)),
            pl.BlockSpec((C_BOX, K), lambda i: (0, 0)),
            pl.BlockSpec((1, C_CLS), lambda i: (0, 0)),
            pl.BlockSpec((1, C_BOX), lambda i: (0, 0)),
        ],
        out_specs=[
            pl.BlockSpec(memory_space=pltpu.MemorySpace.HBM),
            pl.BlockSpec(memory_space=pltpu.MemorySpace.HBM),
        ],
        out_shape=[
            jax.ShapeDtypeStruct((N, C_CLS), jnp.float32),
            jax.ShapeDtypeStruct((N, C_BOX), jnp.float32),
        ],
        scratch_shapes=[
            pltpu.VMEM((NBUF, BN, C_CLS), jnp.float32),
            pltpu.VMEM((NBUF, BN, C_BOX), jnp.float32),
            pltpu.SemaphoreType.DMA((NBUF,)),
            pltpu.SemaphoreType.DMA((NBUF,)),
        ],
        compiler_params=pltpu.CompilerParams(
            dimension_semantics=("arbitrary",),
        ),
    )(x, W_cls, W_bbox, bc, bb)
    return scores, deltas


# speedup vs baseline: 2.5714x; 1.0013x over previous
import jax
import jax.numpy as jnp
from jax import lax
from jax.experimental import pallas as pl
from jax.experimental.pallas import tpu as pltpu

N = 20000
K = 1024
C_CLS = 82
C_BOX = 324
BN = 2000
STEPS = N // BN
NBUF = 4

_DNUMS = (((1,), (1,)), ((), ()))


def _heads_kernel(x_ref, wc_ref, wb_ref, bc_ref, bb_ref, s_hbm, d_hbm,
                  s_scr, d_scr, s_sem, d_sem, t_sem):
    i = pl.program_id(0)
    slot = lax.rem(i, NBUF)

    def s_copy(step, sl):
        return pltpu.make_async_copy(
            s_scr.at[sl], s_hbm.at[pl.ds(step * BN, BN), :], s_sem.at[sl])

    def d_copy(step, sl):
        return pltpu.make_async_copy(
            d_scr.at[sl, :, 0:256], d_hbm.at[pl.ds(step * BN, BN), 0:256],
            d_sem.at[sl])

    def t_copy(step, sl):
        return pltpu.make_async_copy(
            d_scr.at[sl, :, 256:324], d_hbm.at[pl.ds(step * BN, BN), 256:324],
            t_sem.at[sl])

    @pl.when(i >= NBUF)
    def _reclaim():
        s_copy(i - NBUF, slot).wait()
        d_copy(i - NBUF, slot).wait()
        t_copy(i - NBUF, slot).wait()

    xb = x_ref[...].astype(jnp.bfloat16)
    wc = wc_ref[...].astype(jnp.bfloat16)
    wb = wb_ref[...].astype(jnp.bfloat16)
    s = lax.dot_general(xb, wc, _DNUMS, preferred_element_type=jnp.float32)
    d = lax.dot_general(xb, wb, _DNUMS, preferred_element_type=jnp.float32)
    s_scr[slot] = s + bc_ref[...]
    d_scr[slot] = d + bb_ref[...]

    s_copy(i, slot).start()
    d_copy(i, slot).start()
    t_copy(i, slot).start()

    @pl.when(i == STEPS - 1)
    def _drain():
        for back in range(min(NBUF, STEPS) - 1, -1, -1):
            step = STEPS - 1 - back
            s_copy(step, lax.rem(step, NBUF)).wait()
            d_copy(step, lax.rem(step, NBUF)).wait()
            t_copy(step, lax.rem(step, NBUF)).wait()


def kernel(x, W_cls, b_cls, W_bbox, b_bbox):
    if x.ndim > 2:
        x = x.reshape(x.shape[0], -1)
    bc = b_cls.reshape(1, C_CLS)
    bb = b_bbox.reshape(1, C_BOX)

    scores, deltas = pl.pallas_call(
        _heads_kernel,
        grid=(STEPS,),
        in_specs=[
            pl.BlockSpec((BN, K), lambda i: (i, 0)),
            pl.BlockSpec((C_CLS, K), lambda i: (0, 0)),
            pl.BlockSpec((C_BOX, K), lambda i: (0, 0)),
            pl.BlockSpec((1, C_CLS), lambda i: (0, 0)),
            pl.BlockSpec((1, C_BOX), lambda i: (0, 0)),
        ],
        out_specs=[
            pl.BlockSpec(memory_space=pltpu.MemorySpace.HBM),
            pl.BlockSpec(memory_space=pltpu.MemorySpace.HBM),
        ],
        out_shape=[
            jax.ShapeDtypeStruct((N, C_CLS), jnp.float32),
            jax.ShapeDtypeStruct((N, C_BOX), jnp.float32),
        ],
        scratch_shapes=[
            pltpu.VMEM((NBUF, BN, C_CLS), jnp.float32),
            pltpu.VMEM((NBUF, BN, C_BOX), jnp.float32),
            pltpu.SemaphoreType.DMA((NBUF,)),
            pltpu.SemaphoreType.DMA((NBUF,)),
            pltpu.SemaphoreType.DMA((NBUF,)),
        ],
        compiler_params=pltpu.CompilerParams(
            dimension_semantics=("arbitrary",),
        ),
    )(x, W_cls, W_bbox, bc, bb)
    return scores, deltas


# speedup vs baseline: 2.5906x; 1.0075x over previous
import jax
import jax.numpy as jnp
from jax import lax
from jax.experimental import pallas as pl
from jax.experimental.pallas import tpu as pltpu

N = 20000
K = 1024
C_CLS = 82
C_BOX = 324
BN = 2000

_DNUMS = (((1,), (1,)), ((), ()))


def _heads_kernel(x_ref, wc_ref, wb_ref, bc_ref, bb_ref, s_ref, d_ref):
    xb = x_ref[...].astype(jnp.bfloat16)
    wc = wc_ref[...].astype(jnp.bfloat16)
    wb = wb_ref[...].astype(jnp.bfloat16)
    s = lax.dot_general(xb, wc, _DNUMS, preferred_element_type=jnp.float32)
    d = lax.dot_general(xb, wb, _DNUMS, preferred_element_type=jnp.float32)
    s_ref[...] = s + bc_ref[...]
    d_ref[...] = d + bb_ref[...]


def kernel(x, W_cls, b_cls, W_bbox, b_bbox):
    if x.ndim > 2:
        x = x.reshape(x.shape[0], -1)
    bc = b_cls.reshape(1, C_CLS)
    bb = b_bbox.reshape(1, C_BOX)

    grid = (N // BN,)
    scores, deltas = pl.pallas_call(
        _heads_kernel,
        grid=grid,
        in_specs=[
            pl.BlockSpec((BN, K), lambda i: (i, 0)),
            pl.BlockSpec((C_CLS, K), lambda i: (0, 0)),
            pl.BlockSpec((C_BOX, K), lambda i: (0, 0)),
            pl.BlockSpec((1, C_CLS), lambda i: (0, 0)),
            pl.BlockSpec((1, C_BOX), lambda i: (0, 0)),
        ],
        out_specs=[
            pl.BlockSpec((BN, C_CLS), lambda i: (i, 0)),
            pl.BlockSpec((BN, C_BOX), lambda i: (i, 0)),
        ],
        out_shape=[
            jax.ShapeDtypeStruct((N, C_CLS), jnp.float32),
            jax.ShapeDtypeStruct((N, C_BOX), jnp.float32),
        ],
        compiler_params=pltpu.CompilerParams(
            dimension_semantics=("arbitrary",),
        ),
    )(x, W_cls, W_bbox, bc, bb)
    return scores, deltas
